# baseline (speedup 1.0000x reference)
import jax
import jax.numpy as jnp
from jax.experimental import pallas as pl
from jax.experimental.pallas import tpu as pltpu

_N = 4096
_D = 64
_B = 512
_NB = _N // _B
_EPS2 = 121.0
_MINPTS = 10.0
_ITERS = 64
_SENT = float(_N)
_BIG = float(_N + 1)
_K = 8192.0


def _dbscan_body(pts_ref, ptst_ref, sqc_ref, sqr_ref, out_ref,
                 adj_ref, degc_ref, vec_a_ref, vec_b_ref, rankc_ref):
    ptst = ptst_ref[...]
    sqr = sqr_ref[...]
    ones_col = jnp.ones((_N, 1), jnp.float8_e4m3fn)
    ones_row = jnp.ones((1, _B), jnp.float8_e4m3fn)

    def build(ib, deg_row):
        pb = pts_ref[pl.ds(ib * _B, _B), :]
        g2 = jax.lax.dot_general(pb, ptst, (((1,), (0,)), ((), ())),
                                 preferred_element_type=jnp.float32)
        sqc_b = sqc_ref[pl.ds(ib * _B, _B), :]
        adj = g2 - (sqc_b + sqr) > -_EPS2
        adj_ref[pl.ds(ib * _B, _B), :] = adj.astype(jnp.int8)
        a8 = adj.astype(jnp.float8_e4m3fn)
        degc_ref[pl.ds(ib * _B, _B), :] = jax.lax.dot_general(
            a8, ones_col, (((1,), (0,)), ((), ())),
            preferred_element_type=jnp.float32)
        return deg_row + jax.lax.dot_general(
            ones_row, a8, (((1,), (0,)), ((), ())),
            preferred_element_type=jnp.float32)

    deg_row = jax.lax.fori_loop(0, _NB, build, jnp.zeros((1, _N), jnp.float32))

    core_row = deg_row >= _MINPTS
    core_col = degc_ref[...] >= _MINPTS

    iota_row = jax.lax.broadcasted_iota(jnp.int32, (1, _N), 1).astype(jnp.float32)
    iota_col = jax.lax.broadcasted_iota(jnp.int32, (_N, 1), 0).astype(jnp.float32)

    comp_row0 = jnp.where(core_row, iota_row, _SENT)
    comp_col0 = jnp.where(core_col, iota_col, _SENT)

    def cond(c):
        it, _, _, changed = c
        return jnp.logical_and(changed, it < _ITERS)

    def sweep(c):
        it, comp_row, comp_col, _ = c
        vec_a_ref[...] = jnp.where(core_col, _K - comp_col, 0.0)

        def blk(ib, carry):
            v_row, mx_row = carry
            ab = adj_ref[pl.ds(ib * _B, _B), :].astype(jnp.float32)
            vc_b = vec_a_ref[pl.ds(ib * _B, _B), :]
            mx_b = jnp.max(ab * v_row, axis=1, keepdims=True)
            vc_n = jnp.where(vc_b > 0.0, jnp.maximum(vc_b, mx_b), 0.0)
            vec_a_ref[pl.ds(ib * _B, _B), :] = vc_n
            mx_row = jnp.maximum(
                mx_row, jnp.max(ab * vc_n, axis=0, keepdims=True))
            v_row = jnp.where(v_row > 0.0, jnp.maximum(v_row, mx_row), 0.0)
            return v_row, mx_row

        v_row0 = jnp.where(core_row, _K - comp_row, 0.0)
        _, mx_row = jax.lax.fori_loop(
            0, _NB, blk, (v_row0, jnp.zeros((1, _N), jnp.float32)))
        comp_row_n = jnp.minimum(comp_row, _K - mx_row)
        comp_col_n = jnp.where(core_col, _K - vec_a_ref[...], comp_col)
        delta = jnp.where(jnp.logical_and(core_row, comp_row_n < comp_row),
                          1.0, 0.0)
        changed = jnp.max(delta) > 0.0
        return it + 1, comp_row_n, comp_col_n, changed

    _, comp_row, comp_col, _ = jax.lax.while_loop(
        cond, sweep, (jnp.int32(0), comp_row0, comp_col0, jnp.bool_(True)))

    vec_a_ref[...] = jnp.where(
        jnp.logical_and(core_col, comp_col == iota_col), 1.0, 0.0)

    def crow(jb, acc):
        ir_b = vec_a_ref[pl.ds(jb * _B, _B), :]
        jidx = (jax.lax.broadcasted_iota(jnp.int32, (_B, 1), 0)
                + jb * _B).astype(jnp.float32)
        mask = (jidx <= comp_row).astype(jnp.float32)
        return acc + jnp.sum(mask * ir_b, axis=0, keepdims=True)

    cluster_row = jax.lax.fori_loop(
        0, _NB, crow, jnp.zeros((1, _N), jnp.float32)) - 1.0

    vec_b_ref[...] = jnp.swapaxes(cluster_row, 0, 1)

    w_row = jnp.where(core_row, _K - cluster_row, 0.0)

    def fin(ib, carry):
        ab = adj_ref[pl.ds(ib * _B, _B), :].astype(jnp.float32)
        m_b = _K - jnp.max(ab * w_row, axis=1, keepdims=True)
        cl_b = vec_b_ref[pl.ds(ib * _B, _B), :]
        kc_b = degc_ref[pl.ds(ib * _B, _B), :] >= _MINPTS
        out_ref[pl.ds(ib * _B, _B), :] = jnp.where(
            kc_b, cl_b, jnp.where(m_b < _BIG, m_b, -1.0))
        return carry

    jax.lax.fori_loop(0, _NB, fin, 0)


def kernel(points):
    pts = points.astype(jnp.float32)
    sq = jnp.sum(pts * pts, axis=1, keepdims=True)
    pts_bf = (pts * 1.4142135623730951).astype(jnp.bfloat16)
    return pl.pallas_call(
        _dbscan_body,
        out_shape=jax.ShapeDtypeStruct((_N, 1), jnp.float32),
        scratch_shapes=[
            pltpu.VMEM((_N, _N), jnp.int8),
            pltpu.VMEM((_N, 1), jnp.float32),
            pltpu.VMEM((_N, 1), jnp.float32),
            pltpu.VMEM((_N, 1), jnp.float32),
            pltpu.VMEM((_N, 1), jnp.float32),
        ],
    )(pts_bf, pts_bf.T, sq, sq.T)


# speedup vs baseline: 1.0391x; 1.0391x over previous
import jax
import jax.numpy as jnp
from jax.experimental import pallas as pl
from jax.experimental.pallas import tpu as pltpu

_N = 4096
_D = 64
_B = 512
_NB = _N // _B
_EPS2 = 121.0
_MINPTS = 10.0
_ITERS = 64
_SENT = float(_N)
_BIG = float(_N + 1)
_K = 8192.0


def _dbscan_body(pts_ref, ptst_ref, sqc_ref, sqr_ref, out_ref,
                 adj_ref, degc_ref, vec_a_ref, vec_b_ref, rankc_ref):
    ptst = ptst_ref[...]
    sqr = sqr_ref[...]

    def build(ib, carry):
        pb = pts_ref[pl.ds(ib * _B, _B), :]
        g2 = jax.lax.dot_general(pb, ptst, (((1,), (0,)), ((), ())),
                                 preferred_element_type=jnp.float32)
        sqc_b = sqc_ref[pl.ds(ib * _B, _B), :]
        ai8 = (g2 - (sqc_b + sqr) > -_EPS2).astype(jnp.int8)
        adj_ref[pl.ds(ib * _B, _B), :] = ai8
        degc_ref[pl.ds(ib * _B, _B), :] = jnp.sum(
            ai8.astype(jnp.float32), axis=1, keepdims=True)
        return carry

    jax.lax.fori_loop(0, _NB, build, 0)
    deg_row = jnp.swapaxes(degc_ref[...], 0, 1)

    core_row = deg_row >= _MINPTS
    core_col = degc_ref[...] >= _MINPTS

    iota_row = jax.lax.broadcasted_iota(jnp.int32, (1, _N), 1).astype(jnp.float32)
    iota_col = jax.lax.broadcasted_iota(jnp.int32, (_N, 1), 0).astype(jnp.float32)

    comp_row0 = jnp.where(core_row, iota_row, _SENT)
    comp_col0 = jnp.where(core_col, iota_col, _SENT)

    def cond(c):
        it, _, _, changed = c
        return jnp.logical_and(changed, it < _ITERS)

    def sweep(c):
        it, comp_row, comp_col, _ = c
        vec_a_ref[...] = jnp.where(core_col, _K - comp_col, 0.0)

        def blk(ib, carry):
            v_row, mx_row = carry
            ab = adj_ref[pl.ds(ib * _B, _B), :].astype(jnp.float32)
            vc_b = vec_a_ref[pl.ds(ib * _B, _B), :]
            mx_b = jnp.max(ab * v_row, axis=1, keepdims=True)
            vc_n = jnp.where(vc_b > 0.0, jnp.maximum(vc_b, mx_b), 0.0)
            vec_a_ref[pl.ds(ib * _B, _B), :] = vc_n
            mx_row = jnp.maximum(
                mx_row, jnp.max(ab * vc_n, axis=0, keepdims=True))
            v_row = jnp.where(v_row > 0.0, jnp.maximum(v_row, mx_row), 0.0)
            return v_row, mx_row

        v_row0 = jnp.where(core_row, _K - comp_row, 0.0)
        _, mx_row = jax.lax.fori_loop(
            0, _NB, blk, (v_row0, jnp.zeros((1, _N), jnp.float32)))
        comp_row_n = jnp.minimum(comp_row, _K - mx_row)
        comp_col_n = jnp.where(core_col, _K - vec_a_ref[...], comp_col)
        delta = jnp.where(jnp.logical_and(core_row, comp_row_n < comp_row),
                          1.0, 0.0)
        changed = jnp.max(delta) > 0.0
        return it + 1, comp_row_n, comp_col_n, changed

    _, comp_row, comp_col, _ = jax.lax.while_loop(
        cond, sweep, (jnp.int32(0), comp_row0, comp_col0, jnp.bool_(True)))

    vec_a_ref[...] = jnp.where(
        jnp.logical_and(core_col, comp_col == iota_col), 1.0, 0.0)

    def crow(jb, acc):
        ir_b = vec_a_ref[pl.ds(jb * _B, _B), :]
        jidx = (jax.lax.broadcasted_iota(jnp.int32, (_B, 1), 0)
                + jb * _B).astype(jnp.float32)
        mask = (jidx <= comp_row).astype(jnp.float32)
        return acc + jnp.sum(mask * ir_b, axis=0, keepdims=True)

    cluster_row = jax.lax.fori_loop(
        0, _NB, crow, jnp.zeros((1, _N), jnp.float32)) - 1.0

    vec_b_ref[...] = jnp.swapaxes(cluster_row, 0, 1)

    w_row = jnp.where(core_row, _K - cluster_row, 0.0)

    def fin(ib, carry):
        ab = adj_ref[pl.ds(ib * _B, _B), :].astype(jnp.float32)
        m_b = _K - jnp.max(ab * w_row, axis=1, keepdims=True)
        cl_b = vec_b_ref[pl.ds(ib * _B, _B), :]
        kc_b = degc_ref[pl.ds(ib * _B, _B), :] >= _MINPTS
        out_ref[pl.ds(ib * _B, _B), :] = jnp.where(
            kc_b, cl_b, jnp.where(m_b < _BIG, m_b, -1.0))
        return carry

    jax.lax.fori_loop(0, _NB, fin, 0)


def kernel(points):
    pts = points.astype(jnp.float32)
    sq = jnp.sum(pts * pts, axis=1, keepdims=True)
    pts_bf = (pts * 1.4142135623730951).astype(jnp.bfloat16)
    return pl.pallas_call(
        _dbscan_body,
        out_shape=jax.ShapeDtypeStruct((_N, 1), jnp.float32),
        scratch_shapes=[
            pltpu.VMEM((_N, _N), jnp.int8),
            pltpu.VMEM((_N, 1), jnp.float32),
            pltpu.VMEM((_N, 1), jnp.float32),
            pltpu.VMEM((_N, 1), jnp.float32),
            pltpu.VMEM((_N, 1), jnp.float32),
        ],
    )(pts_bf, pts_bf.T, sq, sq.T)
